# baseline (speedup 1.0000x reference)
import functools

import jax
import jax.numpy as jnp
from jax import lax
from jax.experimental import pallas as pl
from jax.experimental.pallas import tpu as pltpu
from jax.experimental.pallas import tpu_sc as plsc

NNODES = 10000
NEDGES = 320000
NPAIRS = 50000
DF = 128
DH = 256

NC, NS, L = 2, 16, 16
NW = NC * NS
DHALF = DF // 2
K = 125
NCHUNK = NEDGES // (NW * K)
EPT2 = NEDGES // NS
NCHUNK2 = EPT2 // K
ACCR = 10240
ZPT = ACCR // NS
DR_FULL = 640
DR_LAST = NNODES - 15 * DR_FULL

@functools.lru_cache(maxsize=None)
def _mesh():
  return plsc.VectorSubcoreMesh(core_axis_name="c", subcore_axis_name="s",
                                num_cores=NC, num_subcores=NS)


def _make_agg(with_cnt):
  NBUF = 4
  out_type = [jax.ShapeDtypeStruct((NC, NNODES, DHALF), jnp.float32)]
  scratch = (
      [pltpu.VMEM((NCHUNK2, K), jnp.int32)] * 2 +
      [pltpu.VMEM((K, DHALF), jnp.float32)] * NBUF +
      [pltpu.VMEM_SHARED((ACCR, DHALF), jnp.float32)] +
      [pltpu.SemaphoreType.DMA] * (2 * NBUF)
  )
  if with_cnt:
    out_type.append(jax.ShapeDtypeStruct((NC, NNODES, L), jnp.float32))
    scratch += [
        pltpu.VMEM((K, L), jnp.float32),
        pltpu.VMEM_SHARED((ACCR, L), jnp.float32),
        pltpu.SemaphoreType.DMA,
    ]
  half = NCHUNK2 // 2

  def body(ta_hbm, tb_hbm, src_hbm, dst_hbm, zrow_hbm, zcnt_hbm, ones_hbm,
           *rest):
    if with_cnt:
      out_hbm, cnt_hbm = rest[0], rest[1]
      rest = rest[2:]
    else:
      out_hbm = rest[0]
      rest = rest[1:]
    srcb, dstb = rest[0], rest[1]
    rows = rest[2:2 + NBUF]
    acc = rest[2 + NBUF]
    gsem = rest[3 + NBUF:3 + 2 * NBUF]
    ssem = rest[3 + 2 * NBUF:3 + 3 * NBUF]
    if with_cnt:
      onesb, cacc, csem = rest[3 + 3 * NBUF:]
    c = lax.axis_index("c")
    s = lax.axis_index("s")

    pltpu.sync_copy(zrow_hbm, acc.at[pl.ds(s * ZPT, ZPT)])
    if with_cnt:
      pltpu.sync_copy(zcnt_hbm, cacc.at[pl.ds(s * ZPT, ZPT)])
      pltpu.sync_copy(ones_hbm, onesb)
    pltpu.sync_copy(src_hbm.at[pl.ds(s * NCHUNK2, NCHUNK2)], srcb)
    pltpu.sync_copy(dst_hbm.at[pl.ds(s * NCHUNK2, NCHUNK2)], dstb)
    plsc.subcore_barrier()

    PF = NBUF // 2

    def run(table, count_pred):
      for b in range(PF):
        pltpu.async_copy(table.at[srcb.at[b]], rows[b], gsem[b])

      def group(jj, _):
        for b in range(NBUF):
          j = jj * NBUF + b
          t = j + PF
          bt = (b + PF) % NBUF

          @pl.when(t < NCHUNK2)
          def _(t=t, bt=bt):
            @pl.when(t >= NBUF)
            def _():
              pltpu.make_async_copy(rows[bt], acc.at[dstb.at[t - NBUF]],
                                    ssem[bt]).wait()
            pltpu.async_copy(table.at[srcb.at[t]], rows[bt], gsem[bt])

          pltpu.make_async_copy(table.at[srcb.at[j]], rows[b],
                                gsem[b]).wait()
          pltpu.async_copy(rows[b], acc.at[dstb.at[j]], ssem[b], add=True)
          if with_cnt:
            @pl.when(count_pred(j))
            def _(j=j):
              pltpu.async_copy(onesb, cacc.at[dstb.at[j]], csem, add=True)
        return 0

      lax.fori_loop(0, NCHUNK2 // NBUF, group, 0)
      for b in range(NBUF):
        jl = NCHUNK2 - NBUF + b
        pltpu.make_async_copy(rows[b], acc.at[dstb.at[jl]], ssem[b]).wait()

    @pl.when(c == 0)
    def _():
      run(ta_hbm, lambda j0: j0 < half)

    @pl.when(c == 1)
    def _():
      run(tb_hbm, lambda j0: j0 >= half)

    if with_cnt:
      base_j = c * half

      def cdrain(t, _):
        pltpu.make_async_copy(onesb, cacc.at[dstb.at[base_j + t]],
                              csem).wait()
        return 0

      lax.fori_loop(0, half, cdrain, 0)

    plsc.subcore_barrier()
    @pl.when(s < NS - 1)
    def _():
      pltpu.sync_copy(acc.at[pl.ds(s * DR_FULL, DR_FULL)],
                      out_hbm.at[c, pl.ds(s * DR_FULL, DR_FULL)])
      if with_cnt:
        pltpu.sync_copy(cacc.at[pl.ds(s * DR_FULL, DR_FULL)],
                        cnt_hbm.at[c, pl.ds(s * DR_FULL, DR_FULL)])

    @pl.when(s == NS - 1)
    def _():
      pltpu.sync_copy(acc.at[pl.ds(15 * DR_FULL, DR_LAST)],
                      out_hbm.at[c, pl.ds(15 * DR_FULL, DR_LAST)])
      if with_cnt:
        pltpu.sync_copy(cacc.at[pl.ds(15 * DR_FULL, DR_LAST)],
                        cnt_hbm.at[c, pl.ds(15 * DR_FULL, DR_LAST)])

  return pl.kernel(
      body,
      out_type=tuple(out_type) if with_cnt else out_type[0],
      mesh=_mesh(),
      scratch_types=scratch,
      compiler_params=pltpu.CompilerParams(use_tc_tiling_on_sc=False),
      name=f"sc_edge_agg_halved_cnt{int(with_cnt)}",
  )


def _make_zagg():
  NBUF = 4
  scratch = (
      [pltpu.VMEM((NCHUNK, K), jnp.int32)] * 2 +
      [pltpu.VMEM((K, L), jnp.float32)] * NBUF +
      [pltpu.VMEM_SHARED((ACCR, L), jnp.float32)] +
      [pltpu.SemaphoreType.DMA] * (2 * NBUF)
  )

  def body(z_hbm, src_hbm, dst_hbm, zcnt_hbm, out_hbm, *rest):
    srcb, dstb = rest[0], rest[1]
    rows = rest[2:2 + NBUF]
    acc = rest[2 + NBUF]
    gsem = rest[3 + NBUF:3 + 2 * NBUF]
    ssem = rest[3 + 2 * NBUF:3 + 3 * NBUF]
    c = lax.axis_index("c")
    s = lax.axis_index("s")
    wid = s * NC + c

    pltpu.sync_copy(zcnt_hbm, acc.at[pl.ds(s * ZPT, ZPT)])
    pltpu.sync_copy(src_hbm.at[pl.ds(wid * NCHUNK, NCHUNK)], srcb)
    pltpu.sync_copy(dst_hbm.at[pl.ds(wid * NCHUNK, NCHUNK)], dstb)
    plsc.subcore_barrier()

    pltpu.async_copy(z_hbm.at[srcb.at[0]], rows[0], gsem[0])
    pltpu.async_copy(z_hbm.at[srcb.at[1]], rows[1], gsem[1])

    def group(jj, _):
      for b in range(NBUF):
        j = jj * NBUF + b
        t = j + 2
        bt = (b + 2) % NBUF

        @pl.when(t < NCHUNK)
        def _(t=t, bt=bt):
          @pl.when(t >= NBUF)
          def _():
            pltpu.make_async_copy(rows[bt], acc.at[dstb.at[t - NBUF]],
                                  ssem[bt]).wait()
          pltpu.async_copy(z_hbm.at[srcb.at[t]], rows[bt], gsem[bt])

        pltpu.make_async_copy(z_hbm.at[srcb.at[j]], rows[b],
                              gsem[b]).wait()
        pltpu.async_copy(rows[b], acc.at[dstb.at[j]], ssem[b], add=True)
      return 0

    lax.fori_loop(0, NCHUNK // NBUF, group, 0)
    for b in range(NBUF):
      jl = NCHUNK - NBUF + b
      pltpu.make_async_copy(rows[b], acc.at[dstb.at[jl]], ssem[b]).wait()

    plsc.subcore_barrier()

    @pl.when(s < NS - 1)
    def _():
      pltpu.sync_copy(acc.at[pl.ds(s * DR_FULL, DR_FULL)],
                      out_hbm.at[c, pl.ds(s * DR_FULL, DR_FULL)])

    @pl.when(s == NS - 1)
    def _():
      pltpu.sync_copy(acc.at[pl.ds(15 * DR_FULL, DR_LAST)],
                      out_hbm.at[c, pl.ds(15 * DR_FULL, DR_LAST)])

  return pl.kernel(
      body,
      out_type=jax.ShapeDtypeStruct((NC, NNODES, L), jnp.float32),
      mesh=_mesh(),
      scratch_types=scratch,
      compiler_params=pltpu.CompilerParams(use_tc_tiling_on_sc=False),
      name="sc_scalar_agg",
  )


_zagg = functools.lru_cache(maxsize=None)(_make_zagg)


PPT = 1552
REM_BASE = NW * PPT
REM = NPAIRS - REM_BASE


def _pair_body(s_hbm, m0_hbm, m1_hbm, out_hbm, sv, m0v, m1v, outv,
               m0r, m1r, outr):
  c = lax.axis_index("c")
  s = lax.axis_index("s")
  wid = s * NC + c
  base = wid * PPT
  pltpu.sync_copy(s_hbm, sv)
  pltpu.sync_copy(m0_hbm.at[pl.ds(base, PPT)], m0v)
  pltpu.sync_copy(m1_hbm.at[pl.ds(base, PPT)], m1v)

  def body(v, _):
    i0 = m0v[pl.ds(v * L, L)] * 8
    i1 = m1v[pl.ds(v * L, L)] * 8 + 1
    g = plsc.load_gather(sv, [i0]) + plsc.load_gather(sv, [i1])
    outv[pl.ds(v * L, L)] = 1.0 / (1.0 + jnp.exp(-g))
    return 0

  lax.fori_loop(0, PPT // L, body, 0)
  pltpu.sync_copy(outv, out_hbm.at[pl.ds(base, PPT)])

  @pl.when(wid == 0)
  def _():
    pltpu.sync_copy(m0_hbm.at[pl.ds(REM_BASE, REM)], m0r)
    pltpu.sync_copy(m1_hbm.at[pl.ds(REM_BASE, REM)], m1r)

    def rbody(v, _):
      i0 = m0r[pl.ds(v * L, L)] * 8
      i1 = m1r[pl.ds(v * L, L)] * 8 + 1
      g = plsc.load_gather(sv, [i0]) + plsc.load_gather(sv, [i1])
      outr[pl.ds(v * L, L)] = 1.0 / (1.0 + jnp.exp(-g))
      return 0

    lax.fori_loop(0, REM // L, rbody, 0)
    pltpu.sync_copy(outr, out_hbm.at[pl.ds(REM_BASE, REM)])


@functools.lru_cache(maxsize=None)
def _pair_kernel():
  return pl.kernel(
      _pair_body,
      out_type=jax.ShapeDtypeStruct((NPAIRS,), jnp.float32),
      mesh=_mesh(),
      compiler_params=pltpu.CompilerParams(use_tc_tiling_on_sc=False,
                                           needs_layout_passes=False),
      scratch_types=[
          pltpu.VMEM((NNODES * 8,), jnp.float32),
          pltpu.VMEM((PPT,), jnp.int32),
          pltpu.VMEM((PPT,), jnp.int32),
          pltpu.VMEM((PPT,), jnp.float32),
          pltpu.VMEM((REM,), jnp.int32),
          pltpu.VMEM((REM,), jnp.int32),
          pltpu.VMEM((REM,), jnp.float32),
      ],
      name="sc_pair_readout",
  )


BLK = 1000
_DOT = functools.partial(
    lax.dot_general,
    dimension_numbers=(((1,), (1,)), ((), ())),
    preferred_element_type=jnp.float32,
    precision=lax.Precision.HIGHEST,
)


_DOT10 = functools.partial(
    lax.dot_general,
    dimension_numbers=(((1,), (0,)), ((), ())),
    preferred_element_type=jnp.float32,
    precision=lax.Precision.HIGHEST,
)


def _tc1_body(a0, a1, c0, c1, x, w1a, w1b, b1, w1r, w2a, w2b, w2ra, w2rb,
              w3ra, w3rb, b2a, b2b, b3v, z_out, r_out):
  cnt = jnp.maximum(c0[:, 0:1] + c1[:, 0:1], 1.0)
  h = (_DOT(a0[...] / cnt, w1a[...]) + _DOT(a1[...] / cnt, w1b[...])
       + b1[...] + _DOT(x[...], w1r[...]))
  h = jnp.maximum(h, 0.0)
  v = _DOT10(w3ra[...], w2a[...]) + _DOT10(w3rb[...], w2b[...])
  u = _DOT10(w3ra[...], w2ra[...]) + _DOT10(w3rb[...], w2rb[...])
  z = _DOT(h, v)
  r = _DOT(h, u) + _DOT(b2a[...], w3ra[...]) + _DOT(b2b[...], w3rb[...])
  z_out[...] = jnp.concatenate(
      [z, jnp.zeros((z.shape[0], L - 2), jnp.float32)], axis=1)
  r_out[...] = jnp.concatenate(
      [r + b3v[...], jnp.zeros((r.shape[0], 6), jnp.float32)], axis=1)


def _tc2_body(zs0, zs1, c0, c1, r8, s_out):
  cnt = jnp.maximum(c0[:, 0:1] + c1[:, 0:1], 1.0)
  zz = zs0[...] + zs1[...]
  s12 = zz[:, 0:2] / cnt + r8[:, 0:2]
  s_out[...] = jnp.concatenate(
      [s12, jnp.zeros((s12.shape[0], 6), jnp.float32)], axis=1)


def _row_spec(d):
  return pl.BlockSpec((BLK, d), lambda i: (i, 0))


def _full_spec(shape):
  return pl.BlockSpec(shape, lambda i: tuple(0 for _ in shape))


_tc1 = pl.pallas_call(
    _tc1_body,
    grid=(NNODES // BLK,),
    in_specs=[
        _row_spec(DHALF), _row_spec(DHALF), _row_spec(L), _row_spec(L),
        _row_spec(DF),
        _full_spec((DH, DHALF)), _full_spec((DH, DHALF)),
        _full_spec((1, DH)), _full_spec((DH, DF)),
        _full_spec((DHALF, DH)), _full_spec((DHALF, DH)),
        _full_spec((DHALF, DH)), _full_spec((DHALF, DH)),
        _full_spec((2, DHALF)), _full_spec((2, DHALF)),
        _full_spec((1, DHALF)), _full_spec((1, DHALF)),
        _full_spec((1, 2)),
    ],
    out_specs=[_row_spec(L), _row_spec(8)],
    out_shape=[
        jax.ShapeDtypeStruct((NNODES, L), jnp.float32),
        jax.ShapeDtypeStruct((NNODES, 8), jnp.float32),
    ],
)

_tc2 = pl.pallas_call(
    _tc2_body,
    grid=(NNODES // BLK,),
    in_specs=[
        _row_spec(L), _row_spec(L), _row_spec(L), _row_spec(L),
        _row_spec(8),
    ],
    out_specs=pl.BlockSpec((BLK, 8), lambda i: (i, 0)),
    out_shape=jax.ShapeDtypeStruct((NNODES, 8), jnp.float32),
)

_agg = functools.lru_cache(maxsize=None)(_make_agg)


@jax.jit
def kernel(g, features, mask, W1, b1, W1r, W2, b2, W2r, W3, b3):
  src = g[0].reshape(NW * NCHUNK, K)
  dst = g[1].reshape(NW * NCHUNK, K)
  m0 = mask[:, 0]
  m1 = mask[:, 1]

  zrow = jnp.zeros((ZPT, DHALF), jnp.float32)
  zcnt = jnp.zeros((ZPT, L), jnp.float32)
  onesp = jnp.zeros((K, L), jnp.float32).at[:, 0].set(1.0)

  xa = features[:, :DHALF]
  xb = features[:, DHALF:]
  sum1, cnt = _agg(True)(xa, xb, src, dst, zrow, zcnt, onesp)

  w3ra = jnp.stack([W3[0, :DHALF], W3[0, DF:DF + DHALF]])
  w3rb = jnp.stack([W3[0, DHALF:DF], W3[0, DF + DHALF:]])
  b2r = b2.reshape(1, DF)
  b3v = jnp.zeros((1, 2), jnp.float32).at[0, 0].set(b3[0])
  z16, r8 = _tc1(
      sum1[0], sum1[1], cnt[0], cnt[1], features,
      W1[:, :DHALF], W1[:, DHALF:], b1.reshape(1, DH), W1r,
      W2[:DHALF], W2[DHALF:], W2r[:DHALF], W2r[DHALF:],
      w3ra, w3rb, b2r[:, :DHALF], b2r[:, DHALF:], b3v)

  zsum = _zagg()(z16, src, dst, zcnt)
  s = _tc2(zsum[0], zsum[1], cnt[0], cnt[1], r8)

  out = _pair_kernel()(s.reshape(NNODES * 8), m0, m1)
  return out.reshape(NPAIRS, 1)
